# baseline (speedup 1.0000x reference)
import jax
import jax.numpy as jnp
from jax import lax
from jax.experimental import pallas as pl
from jax.experimental.pallas import tpu as pltpu
from jax.experimental.pallas import tpu_sc as plsc

NS, NT, D, DT, K = 10000, 4096, 64, 256, 16
PAD = 128
NW = 32
TPW = NT // NW
CT = 16
NCHUNK = TPW // CT
ROWS = CT * K
EPW = TPW * K


def _dense_body(sft, tf, w1, b1, ws, bs, wl, bl, sup, tt):
    f32 = jnp.float32
    zsrc = jnp.zeros((D, PAD - D - 1), f32)
    wsrc = jnp.concatenate([w1[0:D, :], ws[...], zsrc], axis=1)
    sup[...] = lax.dot_general(sft[...], wsrc, (((0,), (0,)), ((), ())),
                               preferred_element_type=f32)
    ztgt = jnp.zeros((DT, PAD - D - 1), f32)
    wtgt = jnp.concatenate([-w1[D:D + DT, :], wl[...], ztgt], axis=1)
    bias = jnp.concatenate([-b1[...], bl[...] + bs[...],
                            jnp.zeros((PAD - D - 1,), f32)])
    tt[...] = jnp.dot(tf[...], wtgt, preferred_element_type=f32) + bias


def _edge_body(sup_hbm, tt_hbm, edge_hbm, w2_hbm, out_hbm,
               idx_v, gbuf_a, gbuf_b, tt_v, w2_v, out_v, sem_a, sem_b):
    w = lax.axis_index("s") * 2 + lax.axis_index("c")
    pltpu.sync_copy(edge_hbm.at[pl.ds(w * EPW, EPW)], idx_v)
    pltpu.sync_copy(tt_hbm.at[pl.ds(w * TPW, TPW)], tt_v)
    pltpu.sync_copy(w2_hbm, w2_v)
    lane = lax.iota(jnp.int32, 16)
    nd = D // 16
    w2v = [w2_v[pl.ds(i * 16, 16)] for i in range(nd)]

    def issue(c, gbuf, sem):
        pltpu.async_copy(
            sup_hbm.at[idx_v.at[pl.ds(c * ROWS, 128)]],
            gbuf.at[pl.ds(0, 128)], sem)
        pltpu.async_copy(
            sup_hbm.at[idx_v.at[pl.ds(c * ROWS + 128, 128)]],
            gbuf.at[pl.ds(128, 128)], sem)

    def wait(gbuf, sem):
        pltpu.make_async_copy(sup_hbm.at[idx_v.at[pl.ds(0, 128)]],
                              gbuf.at[pl.ds(0, 128)], sem).wait()
        pltpu.make_async_copy(sup_hbm.at[idx_v.at[pl.ds(0, 128)]],
                              gbuf.at[pl.ds(128, 128)], sem).wait()

    def compute(c, gbuf):
        tpb_vec = plsc.load_gather(tt_v, [c * CT + lane,
                                          jnp.full((16,), D, jnp.int32)])

        zero16 = jnp.zeros((16,), jnp.float32)

        @plsc.parallel_loop(0, CT, 1, unroll=2,
                            carry=(zero16, jnp.ones((16,), jnp.float32)))
        def tgt_body(t, carry2):
            num_acc, den_acc = carry2
            ct = c * CT + t
            ttv = [tt_v[ct, pl.ds(i * 16, 16)] for i in range(nd)]
            score = jnp.zeros((16,), jnp.float32)
            for k in range(K):
                row = t * K + k
                p = (jnp.maximum(gbuf[row, pl.ds(0, 16)], ttv[0]) * w2v[0]
                     + jnp.maximum(gbuf[row, pl.ds(16, 16)], ttv[1]) * w2v[1]
                     + jnp.maximum(gbuf[row, pl.ds(32, 16)], ttv[2]) * w2v[2]
                     + jnp.maximum(gbuf[row, pl.ds(48, 16)], ttv[3]) * w2v[3])
                score = jnp.where(lane == k, jnp.sum(p), score)
            e = jnp.exp(score)
            rows = t * K + lane
            spg = plsc.load_gather(gbuf, [rows, jnp.full((16,), D, jnp.int32)])
            num = jnp.sum(e * spg)
            den = jnp.sum(e)
            return (jnp.where(lane == t, num, num_acc),
                    jnp.where(lane == t, den, den_acc))

        num_vec, den_vec = tgt_body
        out_v[pl.ds(c * CT, CT)] = tpb_vec + num_vec / den_vec

    issue(0, gbuf_a, sem_a)

    def pair_body(cp, carry):
        c0 = 2 * cp
        issue(c0 + 1, gbuf_b, sem_b)
        wait(gbuf_a, sem_a)
        compute(c0, gbuf_a)

        @pl.when(cp < NCHUNK // 2 - 1)
        def _():
            issue(c0 + 2, gbuf_a, sem_a)

        wait(gbuf_b, sem_b)
        compute(c0 + 1, gbuf_b)
        return carry

    lax.fori_loop(0, NCHUNK // 2, pair_body, 0)
    pltpu.sync_copy(out_v, out_hbm.at[pl.ds(w * TPW, TPW)])


def kernel(source_feat, target_feat, edge_src, W1, b1, W2, b2, Ws, bs, Wl, bl):
    f32 = jnp.float32
    sup, tt = pl.pallas_call(
        _dense_body,
        out_shape=(
            jax.ShapeDtypeStruct((NS, PAD), f32),
            jax.ShapeDtypeStruct((NT, PAD), f32),
        ),
    )(source_feat.T, target_feat, W1, b1, Ws, bs, Wl, bl)

    mesh = plsc.VectorSubcoreMesh(core_axis_name="c", subcore_axis_name="s")
    edge_fn = pl.kernel(
        _edge_body,
        out_type=jax.ShapeDtypeStruct((NT,), f32),
        mesh=mesh,
        compiler_params=pltpu.CompilerParams(needs_layout_passes=False),
        scratch_types=[
            pltpu.VMEM((EPW,), jnp.int32),
            pltpu.VMEM((ROWS, PAD), f32),
            pltpu.VMEM((ROWS, PAD), f32),
            pltpu.VMEM((TPW, PAD), f32),
            pltpu.VMEM((D,), f32),
            pltpu.VMEM((TPW,), f32),
            pltpu.SemaphoreType.DMA,
            pltpu.SemaphoreType.DMA,
        ],
    )
    return edge_fn(
        sup,
        tt,
        edge_src.reshape(NT * K),
        W2.reshape(D),
    )


# speedup vs baseline: 1.0669x; 1.0669x over previous
import jax
import jax.numpy as jnp
from jax import lax
from jax.experimental import pallas as pl
from jax.experimental.pallas import tpu as pltpu
from jax.experimental.pallas import tpu_sc as plsc

NS, NT, D, DT, K = 10000, 4096, 64, 256, 16
PAD = 128
NW = 32
TPW = NT // NW
CT = 16
NCHUNK = TPW // CT
ROWS = CT * K
EPW = TPW * K


def _dense_body(sft, tf, w1, b1, ws, bs, wl, bl, sup, tt):
    f32 = jnp.float32
    zsrc = jnp.zeros((D, PAD - D - 1), f32)
    wsrc = jnp.concatenate([w1[0:D, :], ws[...], zsrc], axis=1)
    sup[...] = lax.dot_general(sft[...], wsrc, (((0,), (0,)), ((), ())),
                               preferred_element_type=f32)
    ztgt = jnp.zeros((DT, PAD - D - 1), f32)
    wtgt = jnp.concatenate([-w1[D:D + DT, :], wl[...], ztgt], axis=1)
    bias = jnp.concatenate([-b1[...], bl[...] + bs[...],
                            jnp.zeros((PAD - D - 1,), f32)])
    tt[...] = jnp.dot(tf[...], wtgt, preferred_element_type=f32) + bias


def _edge_body(sup_hbm, tt_hbm, edge_hbm, w2_hbm, out_hbm,
               idx_v, gbuf_a, gbuf_b, tt_v, w2_v, out_v, sem_a, sem_b):
    w = lax.axis_index("s") * 2 + lax.axis_index("c")
    pltpu.sync_copy(edge_hbm.at[pl.ds(w * EPW, EPW)], idx_v)
    pltpu.sync_copy(tt_hbm.at[pl.ds(w * TPW, TPW)], tt_v)
    pltpu.sync_copy(w2_hbm, w2_v)
    lane = lax.iota(jnp.int32, 16)
    nd = D // 16
    w2v = [w2_v[pl.ds(i * 16, 16)] for i in range(nd)]

    def issue(c, gbuf, sem):
        pltpu.async_copy(
            sup_hbm.at[idx_v.at[pl.ds(c * ROWS, 128)]],
            gbuf.at[pl.ds(0, 128)], sem)
        pltpu.async_copy(
            sup_hbm.at[idx_v.at[pl.ds(c * ROWS + 128, 128)]],
            gbuf.at[pl.ds(128, 128)], sem)

    def wait(gbuf, sem):
        pltpu.make_async_copy(sup_hbm.at[idx_v.at[pl.ds(0, 128)]],
                              gbuf.at[pl.ds(0, 128)], sem).wait()
        pltpu.make_async_copy(sup_hbm.at[idx_v.at[pl.ds(0, 128)]],
                              gbuf.at[pl.ds(128, 128)], sem).wait()

    def compute(c, gbuf):
        tpb_vec = plsc.load_gather(tt_v, [c * CT + lane,
                                          jnp.full((16,), D, jnp.int32)])

        def tgt_body(t, carry2):
            num_acc, den_acc = carry2
            ct = c * CT + t
            ttv = [tt_v[ct, pl.ds(i * 16, 16)] for i in range(nd)]
            score = jnp.zeros((16,), jnp.float32)
            for k in range(K):
                row = t * K + k
                p = (jnp.maximum(gbuf[row, pl.ds(0, 16)], ttv[0]) * w2v[0]
                     + jnp.maximum(gbuf[row, pl.ds(16, 16)], ttv[1]) * w2v[1]
                     + jnp.maximum(gbuf[row, pl.ds(32, 16)], ttv[2]) * w2v[2]
                     + jnp.maximum(gbuf[row, pl.ds(48, 16)], ttv[3]) * w2v[3])
                score = jnp.where(lane == k, jnp.sum(p), score)
            e = jnp.exp(score)
            rows = t * K + lane
            spg = plsc.load_gather(gbuf, [rows, jnp.full((16,), D, jnp.int32)])
            num = jnp.sum(e * spg)
            den = jnp.sum(e)
            return (jnp.where(lane == t, num, num_acc),
                    jnp.where(lane == t, den, den_acc))

        zero16 = jnp.zeros((16,), jnp.float32)
        num_vec, den_vec = lax.fori_loop(0, CT, tgt_body,
                                         (zero16, jnp.ones((16,), jnp.float32)))
        out_v[pl.ds(c * CT, CT)] = tpb_vec + num_vec / den_vec

    issue(0, gbuf_a, sem_a)

    def pair_body(cp, carry):
        c0 = 2 * cp
        issue(c0 + 1, gbuf_b, sem_b)
        wait(gbuf_a, sem_a)
        compute(c0, gbuf_a)

        @pl.when(cp < NCHUNK // 2 - 1)
        def _():
            issue(c0 + 2, gbuf_a, sem_a)

        wait(gbuf_b, sem_b)
        compute(c0 + 1, gbuf_b)
        return carry

    lax.fori_loop(0, NCHUNK // 2, pair_body, 0)
    pltpu.sync_copy(out_v, out_hbm.at[pl.ds(w * TPW, TPW)])


def kernel(source_feat, target_feat, edge_src, W1, b1, W2, b2, Ws, bs, Wl, bl):
    f32 = jnp.float32
    sup, tt = pl.pallas_call(
        _dense_body,
        out_shape=(
            jax.ShapeDtypeStruct((NS, PAD), f32),
            jax.ShapeDtypeStruct((NT, PAD), f32),
        ),
    )(source_feat.T, target_feat, W1, b1, Ws, bs, Wl, bl)

    mesh = plsc.VectorSubcoreMesh(core_axis_name="c", subcore_axis_name="s")
    edge_fn = pl.kernel(
        _edge_body,
        out_type=jax.ShapeDtypeStruct((NT,), f32),
        mesh=mesh,
        compiler_params=pltpu.CompilerParams(needs_layout_passes=False),
        scratch_types=[
            pltpu.VMEM((EPW,), jnp.int32),
            pltpu.VMEM((ROWS, PAD), f32),
            pltpu.VMEM((ROWS, PAD), f32),
            pltpu.VMEM((TPW, PAD), f32),
            pltpu.VMEM((D,), f32),
            pltpu.VMEM((TPW,), f32),
            pltpu.SemaphoreType.DMA,
            pltpu.SemaphoreType.DMA,
        ],
    )
    return edge_fn(
        sup,
        tt,
        edge_src.reshape(NT * K),
        W2.reshape(D),
    )
